# Initial kernel scaffold; baseline (speedup 1.0000x reference)
import jax
import jax.numpy as jnp
from jax.experimental import pallas as pl


def kernel(x1, x2, edge_attr1, edge_attr2, matching_idx, W_edge, b_edge, W_node, b_node):
    raise NotImplementedError("write your pallas kernel here")


# speedup vs baseline: 1.1768x; 1.1768x over previous
import functools

import jax
import jax.numpy as jnp
from jax import lax
from jax.experimental import pallas as pl
from jax.experimental.pallas import tpu as pltpu
from jax.experimental.pallas import tpu_sc as plsc

N1 = 10000
N2 = 10000
M = 320000
D = 128
DE = 16

CH = 80
GROUPS = CH // 16


def _edge_scores_sc(x1, x2, ea1, ea2, src, end, in1, in2, ie1, ie2):
    info = plsc.get_sparse_core_info()
    NC, NS = info.num_cores, info.num_subcores
    NW = NC * NS
    epw = M // NW
    nchunks = epw // CH

    mesh = plsc.VectorSubcoreMesh(core_axis_name="c", subcore_axis_name="s")
    fdt = jnp.float32
    idt = jnp.int32

    @functools.partial(
        pl.kernel,
        mesh=mesh,
        compiler_params=pltpu.CompilerParams(
            needs_layout_passes=False, use_tc_tiling_on_sc=False),
        out_type=[
            jax.ShapeDtypeStruct((M,), fdt),
            jax.ShapeDtypeStruct((M,), fdt),
            jax.ShapeDtypeStruct((M,), fdt),
        ],
        scratch_types=[
            pltpu.VMEM((CH,), idt),
            pltpu.VMEM((CH,), idt),
            pltpu.VMEM((CH, D), fdt),
            pltpu.VMEM((CH, D), fdt),
            pltpu.VMEM((CH, DE), fdt),
            pltpu.VMEM((CH, DE), fdt),
            pltpu.VMEM((CH,), fdt),
            pltpu.VMEM((CH,), fdt),
            pltpu.VMEM((CH,), fdt),
            pltpu.VMEM((CH,), fdt),
            pltpu.VMEM((epw,), fdt),
            pltpu.VMEM((epw,), fdt),
            pltpu.VMEM((epw,), fdt),
            pltpu.SemaphoreType.DMA,
        ],
    )
    def k(x1_h, x2_h, ea1_h, ea2_h, src_h, end_h, in1_h, in2_h, ie1_h, ie2_h,
          ecos_h, q_h, ecos2_h,
          src_c, end_c, x1g, x2g, ea1g, ea2g, g1, g2, ge1, ge2,
          ecos_v, q_v, ecos2_v, sem):
        wid = lax.axis_index("s") * NC + lax.axis_index("c")
        wbase = wid * epw
        lane = lax.iota(idt, 16)

        def chunk_body(c, carry):
            base = wbase + c * CH
            pltpu.sync_copy(src_h.at[pl.ds(base, CH)], src_c)
            pltpu.sync_copy(end_h.at[pl.ds(base, CH)], end_c)
            hs = [
                pltpu.async_copy(x1_h.at[src_c], x1g, sem),
                pltpu.async_copy(x2_h.at[end_c], x2g, sem),
                pltpu.async_copy(ea1_h.at[src_c], ea1g, sem),
                pltpu.async_copy(ea2_h.at[end_c], ea2g, sem),
                pltpu.async_copy(in1_h.at[src_c], g1, sem),
                pltpu.async_copy(in2_h.at[end_c], g2, sem),
                pltpu.async_copy(ie1_h.at[src_c], ge1, sem),
                pltpu.async_copy(ie2_h.at[end_c], ge2, sem),
            ]
            for h in hs:
                h.wait()

            for g in range(GROUPS):
                s0 = g * 16
                dot = jnp.zeros((16,), fdt)
                qacc = jnp.zeros((16,), fdt)
                dote = jnp.zeros((16,), fdt)
                for e in range(16):
                    erow = s0 + e
                    acc = jnp.zeros((16,), fdt)
                    for kk in range(D // 16):
                        va = x1g[erow, pl.ds(kk * 16, 16)]
                        vb = x2g[erow, pl.ds(kk * 16, 16)]
                        acc = acc + va * vb
                    ds = jnp.sum(acc)
                    ua = ea1g[erow, :]
                    ub = ea2g[erow, :]
                    diff = ua - ub + 1e-6
                    qs = jnp.sum(diff * diff)
                    es = jnp.sum(ua * ub)
                    sel = lane == e
                    dot = jnp.where(sel, ds, dot)
                    qacc = jnp.where(sel, qs, qacc)
                    dote = jnp.where(sel, es, dote)
                off = c * CH + s0
                ecos_v[pl.ds(off, 16)] = dot * g1[pl.ds(s0, 16)] * g2[pl.ds(s0, 16)]
                q_v[pl.ds(off, 16)] = qacc
                ecos2_v[pl.ds(off, 16)] = dote * ge1[pl.ds(s0, 16)] * ge2[pl.ds(s0, 16)]
            return carry

        lax.fori_loop(0, nchunks, chunk_body, 0)
        pltpu.sync_copy(ecos_v, ecos_h.at[pl.ds(wbase, epw)])
        pltpu.sync_copy(q_v, q_h.at[pl.ds(wbase, epw)])
        pltpu.sync_copy(ecos2_v, ecos2_h.at[pl.ds(wbase, epw)])

    return k(x1, x2, ea1, ea2, src, end, in1, in2, ie1, ie2)


def _linear_body(x_ref, w_ref, b_ref, o_ref):
    o_ref[...] = (
        jnp.dot(x_ref[...], w_ref[...], preferred_element_type=jnp.float32)
        + b_ref[...]
    )


def _linear(x, w, b, rows_per_block=2000):
    n, kdim = x.shape
    odim = w.shape[1]
    grid = n // rows_per_block
    return pl.pallas_call(
        _linear_body,
        grid=(grid,),
        in_specs=[
            pl.BlockSpec((rows_per_block, kdim), lambda i: (i, 0)),
            pl.BlockSpec((kdim, odim), lambda i: (0, 0)),
            pl.BlockSpec((1, odim), lambda i: (0, 0)),
        ],
        out_specs=pl.BlockSpec((rows_per_block, odim), lambda i: (i, 0)),
        out_shape=jax.ShapeDtypeStruct((n, odim), jnp.float32),
    )(x, w, b.reshape(1, odim))


def _seg_max_arg(vals, idx, num_segments):
    maxv = jax.ops.segment_max(vals, idx, num_segments=num_segments)
    is_max = vals == maxv[idx]
    arg = jax.ops.segment_min(
        jnp.where(is_max, jnp.arange(M, dtype=jnp.int32), M), idx,
        num_segments=num_segments)
    arg = jnp.clip(arg, 0, M - 1)
    maxv = jnp.where(jnp.isfinite(maxv), maxv, 0.0)
    return maxv, arg


def kernel(x1, x2, edge_attr1, edge_attr2, matching_idx, W_edge, b_edge,
           W_node, b_node):
    src = matching_idx[0].astype(jnp.int32)
    end = matching_idx[1].astype(jnp.int32)

    inv_n1 = 1.0 / jnp.maximum(jnp.linalg.norm(x1, axis=-1), 1e-8)
    inv_n2 = 1.0 / jnp.maximum(jnp.linalg.norm(x2, axis=-1), 1e-8)
    inv_e1 = 1.0 / jnp.maximum(jnp.linalg.norm(edge_attr1, axis=-1), 1e-8)
    inv_e2 = 1.0 / jnp.maximum(jnp.linalg.norm(edge_attr2, axis=-1), 1e-8)

    e_cos, q, e_cos2 = _edge_scores_sc(
        x1, x2, edge_attr1, edge_attr2, src, end,
        inv_n1, inv_n2, inv_e1, inv_e2)
    e_dis = -jnp.sqrt(q)

    _, argsim1 = _seg_max_arg(e_cos, src, N1)
    _, argsim2 = _seg_max_arg(e_cos, end, N2)
    x2_part = jnp.take(x2, jnp.take(end, argsim1), axis=0)
    x1_part = jnp.take(x1, jnp.take(src, argsim2), axis=0)
    out_edge_attr1 = _linear(
        jnp.concatenate([edge_attr1, x1, x2_part], axis=-1), W_edge, b_edge)
    out_edge_attr2 = _linear(
        jnp.concatenate([edge_attr2, x2, x1_part], axis=-1), W_edge, b_edge)

    dis1, argdis1 = _seg_max_arg(e_dis, src, N1)
    sim1, _ = _seg_max_arg(e_cos2, src, N1)
    dis2, argdis2 = _seg_max_arg(e_dis, end, N2)
    sim2, _ = _seg_max_arg(e_cos2, end, N2)
    ea2_part = jnp.take(edge_attr2, jnp.take(end, argdis1), axis=0)
    ea1_part = jnp.take(edge_attr1, jnp.take(src, argdis2), axis=0)
    out_x1 = _linear(
        jnp.concatenate([x1, edge_attr1, ea2_part, dis1[:, None],
                         sim1[:, None]], axis=-1), W_node, b_node)
    out_x2 = _linear(
        jnp.concatenate([x2, edge_attr2, ea1_part, dis2[:, None],
                         sim2[:, None]], axis=-1), W_node, b_node)

    return (out_x1, out_x2, out_edge_attr1, out_edge_attr2)
